# baseline (speedup 1.0000x reference)
import functools
import jax
import jax.numpy as jnp
from jax import lax
from jax.experimental import pallas as pl
from jax.experimental.pallas import tpu as pltpu
from jax.experimental.pallas import tpu_sc as plsc

D = 64
SCALE = 8.0

NUM_CORES = 2
NUM_SUBCORES = 16
NW = NUM_CORES * NUM_SUBCORES

BATCH = 4096
SEQ = 200
ROWS = BATCH * SEQ
ROWS_PER_W = ROWS // NW
CHUNK = SEQ
NCHUNK = ROWS_PER_W // CHUNK


def _gather_fn():
    mesh = plsc.VectorSubcoreMesh(core_axis_name="c", subcore_axis_name="s")

    @functools.partial(
        pl.kernel,
        out_type=jax.ShapeDtypeStruct((BATCH, SEQ, D), jnp.float32),
        mesh=mesh,
        scratch_types=[
            pltpu.VMEM((CHUNK,), jnp.int32),
            pltpu.VMEM((CHUNK,), jnp.int32),
            pltpu.VMEM((CHUNK, D), jnp.float32),
            pltpu.VMEM((CHUNK, D), jnp.float32),
            pltpu.SemaphoreType.DMA,
            pltpu.SemaphoreType.DMA,
        ],
        compiler_params=pltpu.CompilerParams(
            use_tc_tiling_on_sc=False, needs_layout_passes=False),
    )
    def gather_kernel(idx_hbm, table_hbm, out_hbm,
                      idx0, idx1, rows0, rows1, sem0, sem1):
        wid = lax.axis_index("s") * NUM_CORES + lax.axis_index("c")
        base = wid * ROWS_PER_W
        idxs = (idx0, idx1)
        rows = (rows0, rows1)
        sems = (sem0, sem1)

        def fire(i, slot):
            pltpu.sync_copy(idx_hbm.at[pl.ds(base + i * CHUNK, CHUNK)],
                            idxs[slot])
            pltpu.async_copy(table_hbm.at[idxs[slot]], rows[slot], sems[slot])

        def wait(slot):
            pltpu.make_async_copy(
                table_hbm.at[idxs[slot]], rows[slot], sems[slot]).wait()

        def scale_write(i, slot):
            def row(r, _):
                for j in range(D // 16):
                    sl = (r, pl.ds(j * 16, 16))
                    rows[slot][sl] = rows[slot][sl] * SCALE
                return 0

            lax.fori_loop(0, CHUNK, row, 0)
            pltpu.sync_copy(rows[slot],
                            out_hbm.at[wid * NCHUNK + i])

        fire(0, 0)

        def pair_body(g, _):
            for b in range(2):
                i = 2 * g + b

                @pl.when(i + 1 < NCHUNK)
                def _():
                    fire(i + 1, 1 - b)

                wait(b)
                scale_write(i, b)
            return 0

        lax.fori_loop(0, NCHUNK // 2, pair_body, 0)

    return gather_kernel


_GATHER = _gather_fn()


def kernel(x, table):
    idx = x.reshape(-1).astype(jnp.int32)
    emb = _GATHER(idx, table)
    return jnp.transpose(emb, (1, 0, 2))


# speedup vs baseline: 1.0587x; 1.0587x over previous
import functools
import jax
import jax.numpy as jnp
from jax import lax
from jax.experimental import pallas as pl
from jax.experimental.pallas import tpu as pltpu
from jax.experimental.pallas import tpu_sc as plsc

D = 64
SCALE = 8.0

NUM_CORES = 2
NUM_SUBCORES = 16
NW = NUM_CORES * NUM_SUBCORES

BATCH = 4096
SEQ = 200
BW = BATCH // NW


def _encoder_fn():
    mesh = plsc.VectorSubcoreMesh(core_axis_name="c", subcore_axis_name="s")

    @functools.partial(
        pl.kernel,
        out_type=jax.ShapeDtypeStruct((SEQ, BATCH, D), jnp.float32),
        mesh=mesh,
        scratch_types=[
            pltpu.VMEM((BW, SEQ), jnp.int32),
            pltpu.VMEM((BW,), jnp.int32),
            pltpu.VMEM((BW,), jnp.int32),
            pltpu.VMEM((BW, D), jnp.float32),
            pltpu.VMEM((BW, D), jnp.float32),
            pltpu.SemaphoreType.DMA,
            pltpu.SemaphoreType.DMA,
        ],
        compiler_params=pltpu.CompilerParams(
            use_tc_tiling_on_sc=False, needs_layout_passes=False),
    )
    def enc_kernel(x_hbm, table_hbm, out_hbm,
                   xblk, idx0, idx1, rows0, rows1, sem0, sem1):
        wid = lax.axis_index("s") * NUM_CORES + lax.axis_index("c")
        b0 = wid * BW
        idxs = (idx0, idx1)
        rows = (rows0, rows1)
        sems = (sem0, sem1)

        pltpu.sync_copy(x_hbm.at[pl.ds(b0, BW), :], xblk)

        def extract(s, slot):
            for k in range(BW // 16):
                rids = lax.iota(jnp.int32, 16) + (k * 16)
                cids = jnp.full((16,), 0, jnp.int32) + s
                idxs[slot][pl.ds(k * 16, 16)] = plsc.load_gather(
                    xblk, [rids, cids])

        def fire(slot):
            pltpu.async_copy(table_hbm.at[idxs[slot]], rows[slot], sems[slot])

        def wait(slot):
            pltpu.make_async_copy(
                table_hbm.at[idxs[slot]], rows[slot], sems[slot]).wait()

        def scale_write(s, slot):
            def row(r, _):
                for j in range(D // 16):
                    sl = (r, pl.ds(j * 16, 16))
                    rows[slot][sl] = rows[slot][sl] * SCALE
                return 0

            lax.fori_loop(0, BW, row, 0)
            pltpu.sync_copy(rows[slot], out_hbm.at[s, pl.ds(b0, BW)])

        extract(0, 0)
        fire(0)

        def pair_body(g, _):
            for b in range(2):
                s = 2 * g + b

                @pl.when(s + 1 < SEQ)
                def _():
                    extract(s + 1, 1 - b)
                    fire(1 - b)

                wait(b)
                scale_write(s, b)
            return 0

        lax.fori_loop(0, SEQ // 2, pair_body, 0)

    return enc_kernel


_ENCODER = _encoder_fn()


def kernel(x, table):
    return _ENCODER(x.astype(jnp.int32), table)
